# baseline (speedup 1.0000x reference)
import jax
import jax.numpy as jnp
import numpy as np
from jax.experimental import pallas as pl
from jax.experimental.pallas import tpu as pltpu

_NUM_CLASSES = 1000
_NCROPS = 2
_TEMP_LOGITS = 0.1
_TEMP_TEACHER = 0.05
_WAIT = 0
_RAMP = 100
_NEPOCHS = 200

_B = 16384
_H = _B // _NCROPS
_R = 2048
_NBLK = _B // _R
_NBH = _H // _R

_LOG2E = float(np.log2(np.e))
_SHIFT = 55.0
_CT = _LOG2E / _TEMP_TEACHER
_CS = _LOG2E / _TEMP_LOGITS

_SCHED = np.concatenate((
    np.zeros(_WAIT),
    np.linspace(0.0, 1.0, _RAMP),
    np.ones(_NEPOCHS - _WAIT - _RAMP),
))


def _body(k_ref, s_ref, t_ref, o_ref, key_s, dlt_s, acc_s):
    i = pl.program_id(0)
    tb = jax.lax.rem(i + _NBLK // 2, _NBLK)

    sr = s_ref[...]
    tr = t_ref[...]

    m = jnp.max(tr, axis=1, keepdims=True)
    at_max = tr == m
    s_a = jnp.max(jnp.where(at_max, sr, jnp.float32(-jnp.inf)),
                  axis=1, keepdims=True)
    m2 = jnp.max(jnp.where(at_max, jnp.float32(-jnp.inf), tr),
                 axis=1, keepdims=True)
    e = jnp.exp2((tr - m) * _CT)
    z = jnp.sum(e, axis=1, keepdims=True)
    e2 = jnp.exp2((m2 - m) * _CT)
    w = e2 + 1e-6 * z

    es = jnp.exp2(sr * _CS - (_SHIFT * _LOG2E))
    lse = _SHIFT + jnp.log(jnp.sum(es, axis=1, keepdims=True))
    dot_ps = jnp.sum(e * sr, axis=1, keepdims=True) * ((1.0 / _TEMP_LOGITS) / z)
    rl_soft = lse - dot_ps
    delta = dot_ps - s_a * (1.0 / _TEMP_LOGITS)

    lane = jax.lax.broadcasted_iota(jnp.int32, (_R, _NBLK), 1)
    hit = lane == tb
    key_s[...] = jnp.where(hit, w, key_s[...])
    dlt_s[...] = jnp.where(hit, delta, dlt_s[...])

    @pl.when(i == 0)
    def _init():
        acc_s[0] = 0.0

    acc_s[0] = acc_s[0] + jnp.sum(rl_soft)

    @pl.when(i == _NBLK - 1)
    def _finish():
        k = k_ref[0]
        bits = jax.lax.bitcast_convert_type(key_s[...], jnp.int32)
        dl = dlt_s[...]
        r_iota = jax.lax.broadcasted_iota(jnp.int32, (_R, _NBLK), 0)
        rowid = jax.lax.rem(lane, _NBH) * _R + r_iota

        total = acc_s[0]
        for c in range(_NCROPS):
            mc = (lane // _NBH) == c

            def cnt(pred, mc=mc):
                return jnp.sum(jnp.where(mc & pred, 1, 0))

            def bs_body(it, u):
                cand = u | (jnp.int32(1) << (30 - it))
                return jnp.where(cnt(bits < cand) < k, cand, u)

            u = jax.lax.fori_loop(0, 31, bs_body, jnp.int32(0))
            c_lt = cnt(bits < u)
            need = k - c_lt
            eq = bits == u

            def ib_body(it, lh):
                lo, hi = lh
                mid = (lo + hi) // 2
                ok = cnt(eq & (rowid <= mid)) >= need
                return (jnp.where(ok, lo, mid + 1), jnp.where(ok, mid, hi))

            istar, _ = jax.lax.fori_loop(
                0, 13, ib_body, (jnp.int32(0), jnp.int32(_H - 1)))

            sel = mc & ((bits < u) | (eq & (rowid <= istar)))
            t_c = jnp.sum(jnp.where(sel, dl, 0.0))
            total = total + jnp.where(k > 0, t_c, 0.0)

        o_ref[...] = jnp.full((1, 1), total / (_NCROPS * _H), jnp.float32)


def _loss_call(k_arr, student_output, teacher_output, interpret=False):
    return pl.pallas_call(
        _body,
        grid=(_NBLK,),
        in_specs=[
            pl.BlockSpec(memory_space=pltpu.SMEM),
            pl.BlockSpec((_R, _NUM_CLASSES), lambda i: (i, 0)),
            pl.BlockSpec((_R, _NUM_CLASSES),
                         lambda i: ((i + _NBLK // 2) % _NBLK, 0)),
        ],
        out_specs=pl.BlockSpec((1, 1), lambda i: (0, 0)),
        out_shape=jax.ShapeDtypeStruct((1, 1), jnp.float32),
        scratch_shapes=[
            pltpu.VMEM((_R, _NBLK), jnp.float32),
            pltpu.VMEM((_R, _NBLK), jnp.float32),
            pltpu.SMEM((1,), jnp.float32),
        ],
        interpret=interpret,
    )(k_arr, student_output, teacher_output)


def kernel(student_output, teacher_output, epoch):
    ratio = jnp.asarray(_SCHED)[epoch]
    k = jnp.floor(_H * ratio).astype(jnp.int32)
    out = _loss_call(jnp.reshape(k, (1,)), student_output, teacher_output)
    return out[0, 0]


# speedup vs baseline: 1.0986x; 1.0986x over previous
import jax
import jax.numpy as jnp
import numpy as np
from jax.experimental import pallas as pl
from jax.experimental.pallas import tpu as pltpu

_NUM_CLASSES = 1000
_NCROPS = 2
_TEMP_LOGITS = 0.1
_TEMP_TEACHER = 0.05
_WAIT = 0
_RAMP = 100
_NEPOCHS = 200

_B = 16384
_H = _B // _NCROPS
_R = 1024
_NBLK = _B // _R
_NBH = _H // _R

_LOG2E = float(np.log2(np.e))
_SHIFT = 55.0
_CT = _LOG2E / _TEMP_TEACHER
_CS = _LOG2E / _TEMP_LOGITS

_SCHED = np.concatenate((
    np.zeros(_WAIT),
    np.linspace(0.0, 1.0, _RAMP),
    np.ones(_NEPOCHS - _WAIT - _RAMP),
))


def _body(k_ref, s_ref, t_ref, o_ref, key_s, dlt_s, acc_s):
    i = pl.program_id(0)
    tb = jax.lax.rem(i + _NBLK // 2, _NBLK)

    sr = s_ref[...]
    tr = t_ref[...]

    m = jnp.max(tr, axis=1, keepdims=True)
    at_max = tr == m
    s_a = jnp.max(jnp.where(at_max, sr, jnp.float32(-jnp.inf)),
                  axis=1, keepdims=True)
    m2 = jnp.max(jnp.where(at_max, jnp.float32(-jnp.inf), tr),
                 axis=1, keepdims=True)
    e = jnp.exp2((tr - m) * _CT)
    z = jnp.sum(e, axis=1, keepdims=True)
    e2 = jnp.exp2((m2 - m) * _CT)
    w = e2 + 1e-6 * z

    es = jnp.exp2(sr * _CS - (_SHIFT * _LOG2E))
    lse = _SHIFT + jnp.log(jnp.sum(es, axis=1, keepdims=True))
    dot_ps = jnp.sum(e * sr, axis=1, keepdims=True) * ((1.0 / _TEMP_LOGITS) / z)
    rl_soft = lse - dot_ps
    delta = dot_ps - s_a * (1.0 / _TEMP_LOGITS)

    lane = jax.lax.broadcasted_iota(jnp.int32, (_R, _NBLK), 1)
    hit = lane == tb
    key_s[...] = jnp.where(hit, w, key_s[...])
    dlt_s[...] = jnp.where(hit, delta, dlt_s[...])

    @pl.when(i == 0)
    def _init():
        acc_s[0] = 0.0

    acc_s[0] = acc_s[0] + jnp.sum(rl_soft)

    @pl.when(i == _NBLK - 1)
    def _finish():
        k = k_ref[0]
        bits = jax.lax.bitcast_convert_type(key_s[...], jnp.int32)
        dl = dlt_s[...]
        r_iota = jax.lax.broadcasted_iota(jnp.int32, (_R, _NBLK), 0)
        rowid = jax.lax.rem(lane, _NBH) * _R + r_iota

        total = acc_s[0]
        for c in range(_NCROPS):
            mc = (lane // _NBH) == c

            def cnt(pred, mc=mc):
                return jnp.sum(jnp.where(mc & pred, 1, 0))

            def bs_body(it, u):
                cand = u | (jnp.int32(1) << (30 - it))
                return jnp.where(cnt(bits < cand) < k, cand, u)

            u = jax.lax.fori_loop(0, 31, bs_body, jnp.int32(0))
            c_lt = cnt(bits < u)
            need = k - c_lt
            eq = bits == u

            def ib_body(it, lh):
                lo, hi = lh
                mid = (lo + hi) // 2
                ok = cnt(eq & (rowid <= mid)) >= need
                return (jnp.where(ok, lo, mid + 1), jnp.where(ok, mid, hi))

            istar, _ = jax.lax.fori_loop(
                0, 13, ib_body, (jnp.int32(0), jnp.int32(_H - 1)))

            sel = mc & ((bits < u) | (eq & (rowid <= istar)))
            t_c = jnp.sum(jnp.where(sel, dl, 0.0))
            total = total + jnp.where(k > 0, t_c, 0.0)

        o_ref[...] = jnp.full((1, 1), total / (_NCROPS * _H), jnp.float32)


def kernel(student_output, teacher_output, epoch):
    ratio = jnp.asarray(_SCHED)[epoch]
    k = jnp.floor(_H * ratio).astype(jnp.int32)
    out = pl.pallas_call(
        _body,
        grid=(_NBLK,),
        in_specs=[
            pl.BlockSpec(memory_space=pltpu.SMEM),
            pl.BlockSpec((_R, _NUM_CLASSES), lambda i: (i, 0)),
            pl.BlockSpec((_R, _NUM_CLASSES),
                         lambda i: ((i + _NBLK // 2) % _NBLK, 0)),
        ],
        out_specs=pl.BlockSpec((1, 1), lambda i: (0, 0)),
        out_shape=jax.ShapeDtypeStruct((1, 1), jnp.float32),
        scratch_shapes=[
            pltpu.VMEM((_R, _NBLK), jnp.float32),
            pltpu.VMEM((_R, _NBLK), jnp.float32),
            pltpu.SMEM((1,), jnp.float32),
        ],
    )(jnp.reshape(k, (1,)), student_output, teacher_output)
    return out[0, 0]
